# baseline (speedup 1.0000x reference)
import functools

import jax
import jax.numpy as jnp
from jax import lax
from jax.experimental import pallas as pl
from jax.experimental.pallas import tpu as pltpu
from jax.experimental.pallas import tpu_sc as plsc

_N_TIPS = 1000
_N_NODES = 2 * _N_TIPS - 2
_N_BRANCHES = 2 * _N_TIPS - 3
_HIDDEN = 16
_MC = 16
_ROWS = _MC * _N_NODES
_PROJ_BLK = 1184
_PAD_B = 2000
_B_TOTAL = _MC * _PAD_B
_NUM_WORKERS = 32
_B_PER_W = _B_TOTAL // _NUM_WORKERS
_EPI_BLK = 1000


def _proj_body(h_ref, w_ref, p1_ref, p2_ref):
    p = jnp.dot(h_ref[0].astype(jnp.bfloat16), w_ref[...].astype(jnp.bfloat16),
                preferred_element_type=jnp.float32)
    p1_ref[0] = p[:, :_HIDDEN]
    p2_ref[0] = p[:, _HIDDEN:]


def _project(h3d, w1s):
    blk_n = 400
    return pl.pallas_call(
        _proj_body,
        grid=(_MC, _PAD_B // blk_n),
        compiler_params=pltpu.CompilerParams(
            dimension_semantics=("parallel", "parallel")),
        in_specs=[
            pl.BlockSpec((1, blk_n, _N_TIPS), lambda i, j: (i, j, 0)),
            pl.BlockSpec((_N_TIPS, 2 * _HIDDEN), lambda i, j: (0, 0)),
        ],
        out_specs=[
            pl.BlockSpec((1, blk_n, _HIDDEN), lambda i, j: (i, j, 0)),
            pl.BlockSpec((1, blk_n, _HIDDEN), lambda i, j: (i, j, 0)),
        ],
        out_shape=[
            jax.ShapeDtypeStruct((_MC, _PAD_B, _HIDDEN), jnp.float32),
            jax.ShapeDtypeStruct((_MC, _PAD_B, _HIDDEN), jnp.float32),
        ],
    )(h3d, w1s)


def _sc_gather(table, idx):
    mesh = plsc.VectorSubcoreMesh(core_axis_name="c", subcore_axis_name="s")

    @functools.partial(
        pl.kernel,
        mesh=mesh,
        out_type=jax.ShapeDtypeStruct((_B_TOTAL, _HIDDEN), jnp.float32),
        scratch_types=[
            pltpu.VMEM((_B_PER_W,), jnp.int32),
            pltpu.VMEM((_B_PER_W, _HIDDEN), jnp.float32),
            pltpu.SemaphoreType.DMA,
        ],
        compiler_params=pltpu.CompilerParams(use_tc_tiling_on_sc=False),
    )
    def k(table_hbm, idx_hbm, out_hbm, idx_v, rows_v, sem):
        wid = lax.axis_index("s") * 2 + lax.axis_index("c")
        base = wid * _B_PER_W
        pltpu.sync_copy(idx_hbm.at[pl.ds(base, _B_PER_W)], idx_v)
        pltpu.async_copy(table_hbm.at[idx_v], rows_v, sem).wait()
        pltpu.sync_copy(rows_v, out_hbm.at[pl.ds(base, _B_PER_W)])

    return k(table, idx)


def _epi_body(p1_ref, g_ref, b1_ref, w2_ref, shift_ref, o_ref):
    z = p1_ref[...] + g_ref[...] + b1_ref[...]
    z = jnp.where(z > 0, z, jnp.exp(jnp.minimum(z, 0.0)) - 1.0)
    out = jnp.dot(z, w2_ref[...], preferred_element_type=jnp.float32)
    o_ref[...] = out + shift_ref[...]


def _epilogue(p1c, g, b1r, w2, shift):
    return pl.pallas_call(
        _epi_body,
        grid=(_B_TOTAL // _EPI_BLK,),
        in_specs=[
            pl.BlockSpec((_EPI_BLK, _HIDDEN), lambda i: (i, 0)),
            pl.BlockSpec((_EPI_BLK, _HIDDEN), lambda i: (i, 0)),
            pl.BlockSpec((1, _HIDDEN), lambda i: (0, 0)),
            pl.BlockSpec((_HIDDEN, 2), lambda i: (0, 0)),
            pl.BlockSpec((1, 2), lambda i: (0, 0)),
        ],
        out_specs=pl.BlockSpec((_EPI_BLK, 2), lambda i: (i, 0)),
        out_shape=jax.ShapeDtypeStruct((_B_TOTAL, 2), jnp.float32),
    )(p1c, g, b1r, w2, shift)


def kernel(node_features, edge_indexes, W1, b1, W2, b2, lscale_mu, lsigma):
    w1s = jnp.concatenate([W1[:_N_TIPS], W1[_N_TIPS:]], axis=1)

    p1, p2 = _project(node_features, w1s)

    pidx = edge_indexes[:, :-1, 0]
    gidx = pidx + (jnp.arange(_MC, dtype=jnp.int32) * _PAD_B)[:, None]
    gidx = jnp.pad(gidx, ((0, 0), (0, _PAD_B - _N_BRANCHES)))
    idx_flat = gidx.reshape(_B_TOTAL)

    g = _sc_gather(p2.reshape(_B_TOTAL, _HIDDEN), idx_flat)

    p1c = p1.reshape(_B_TOTAL, _HIDDEN)

    b1r = b1.reshape(1, _HIDDEN)
    shift = jnp.stack([b2[0] + lscale_mu, b2[1] + lsigma]).reshape(1, 2)

    out = _epilogue(p1c, g, b1r, W2, shift)
    out3 = out.reshape(_MC, _PAD_B, 2)
    mu = out3[:, :_N_BRANCHES, 0]
    ls = out3[:, :_N_BRANCHES, 1]
    return (mu, ls)


# speedup vs baseline: 1.1447x; 1.1447x over previous
import functools

import jax
import jax.numpy as jnp
from jax import lax
from jax.experimental import pallas as pl
from jax.experimental.pallas import tpu as pltpu
from jax.experimental.pallas import tpu_sc as plsc

_N_TIPS = 1000
_N_NODES = 2 * _N_TIPS - 2
_N_BRANCHES = 2 * _N_TIPS - 3
_HIDDEN = 16
_MC = 16
_ROWS = _MC * _N_NODES
_PROJ_BLK = 1184
_PAD_B = 2000
_B_TOTAL = _MC * _PAD_B
_NUM_WORKERS = 32
_B_PER_W = _B_TOTAL // _NUM_WORKERS
_EPI_BLK = 1000


def _proj_body(h_ref, w_ref, p1_ref, p2_ref):
    p = jnp.dot(h_ref[0].astype(jnp.bfloat16), w_ref[...].astype(jnp.bfloat16),
                preferred_element_type=jnp.float32)
    p1_ref[0] = p[:, :_HIDDEN]
    p2_ref[0] = p[:, _HIDDEN:]


def _project(h3d, w1s):
    blk_n = _PAD_B
    return pl.pallas_call(
        _proj_body,
        grid=(_MC, _PAD_B // blk_n),
        compiler_params=pltpu.CompilerParams(
            dimension_semantics=("parallel", "parallel")),
        in_specs=[
            pl.BlockSpec((1, blk_n, _N_TIPS), lambda i, j: (i, j, 0)),
            pl.BlockSpec((_N_TIPS, 2 * _HIDDEN), lambda i, j: (0, 0)),
        ],
        out_specs=[
            pl.BlockSpec((1, blk_n, _HIDDEN), lambda i, j: (i, j, 0)),
            pl.BlockSpec((1, blk_n, _HIDDEN), lambda i, j: (i, j, 0)),
        ],
        out_shape=[
            jax.ShapeDtypeStruct((_MC, _PAD_B, _HIDDEN), jnp.float32),
            jax.ShapeDtypeStruct((_MC, _PAD_B, _HIDDEN), jnp.float32),
        ],
    )(h3d, w1s)


def _sc_gather(table, idx):
    mesh = plsc.VectorSubcoreMesh(core_axis_name="c", subcore_axis_name="s")

    @functools.partial(
        pl.kernel,
        mesh=mesh,
        out_type=jax.ShapeDtypeStruct((_B_TOTAL, _HIDDEN), jnp.float32),
        scratch_types=[
            pltpu.VMEM((_B_PER_W,), jnp.int32),
            pltpu.VMEM((_B_PER_W, _HIDDEN), jnp.float32),
            pltpu.SemaphoreType.DMA,
        ],
        compiler_params=pltpu.CompilerParams(use_tc_tiling_on_sc=False),
    )
    def k(table_hbm, idx_hbm, out_hbm, idx_v, rows_v, sem):
        wid = lax.axis_index("s") * 2 + lax.axis_index("c")
        base = wid * _B_PER_W
        pltpu.sync_copy(idx_hbm.at[pl.ds(base, _B_PER_W)], idx_v)
        pltpu.async_copy(table_hbm.at[idx_v], rows_v, sem).wait()
        pltpu.sync_copy(rows_v, out_hbm.at[pl.ds(base, _B_PER_W)])

    return k(table, idx)


def _epi_body(p1_ref, g_ref, b1_ref, w2_ref, shift_ref, o_ref):
    z = p1_ref[...] + g_ref[...] + b1_ref[...]
    z = jnp.where(z > 0, z, jnp.exp(jnp.minimum(z, 0.0)) - 1.0)
    out = jnp.dot(z, w2_ref[...], preferred_element_type=jnp.float32)
    o_ref[...] = out + shift_ref[...]


def _epilogue(p1c, g, b1r, w2, shift):
    return pl.pallas_call(
        _epi_body,
        grid=(_B_TOTAL // _EPI_BLK,),
        in_specs=[
            pl.BlockSpec((_EPI_BLK, _HIDDEN), lambda i: (i, 0)),
            pl.BlockSpec((_EPI_BLK, _HIDDEN), lambda i: (i, 0)),
            pl.BlockSpec((1, _HIDDEN), lambda i: (0, 0)),
            pl.BlockSpec((_HIDDEN, 2), lambda i: (0, 0)),
            pl.BlockSpec((1, 2), lambda i: (0, 0)),
        ],
        out_specs=pl.BlockSpec((_EPI_BLK, 2), lambda i: (i, 0)),
        out_shape=jax.ShapeDtypeStruct((_B_TOTAL, 2), jnp.float32),
    )(p1c, g, b1r, w2, shift)


def kernel(node_features, edge_indexes, W1, b1, W2, b2, lscale_mu, lsigma):
    w1s = jnp.concatenate([W1[:_N_TIPS], W1[_N_TIPS:]], axis=1)

    p1, p2 = _project(node_features, w1s)

    pidx = edge_indexes[:, :-1, 0]
    gidx = pidx + (jnp.arange(_MC, dtype=jnp.int32) * _PAD_B)[:, None]
    gidx = jnp.pad(gidx, ((0, 0), (0, _PAD_B - _N_BRANCHES)))
    idx_flat = gidx.reshape(_B_TOTAL)

    g = _sc_gather(p2.reshape(_B_TOTAL, _HIDDEN), idx_flat)

    p1c = p1.reshape(_B_TOTAL, _HIDDEN)

    b1r = b1.reshape(1, _HIDDEN)
    shift = jnp.stack([b2[0] + lscale_mu, b2[1] + lsigma]).reshape(1, 2)

    out = _epilogue(p1c, g, b1r, W2, shift)
    out3 = out.reshape(_MC, _PAD_B, 2)
    mu = out3[:, :_N_BRANCHES, 0]
    ls = out3[:, :_N_BRANCHES, 1]
    return (mu, ls)
